# baseline (speedup 1.0000x reference)
import functools

import jax
import jax.numpy as jnp
from jax import lax
from jax.experimental import pallas as pl
from jax.experimental.pallas import tpu as pltpu
from jax.experimental.pallas import tpu_sc as plsc

BATCH = 16384
NF = 64

_info = plsc.get_sparse_core_info()
_NC, _NS = _info.num_cores, _info.num_subcores
_NW = _NC * _NS
_BPW = BATCH // _NW


def _gather_body(U_hbm, M_hbm, users_hbm, movies_hbm, eu_hbm, em_hbm,
                 users_v, movies_v, rows_u, rows_m, sem):
    wid = lax.axis_index("s") * _NC + lax.axis_index("c")
    base = wid * _BPW
    pltpu.sync_copy(users_hbm.at[pl.ds(base, _BPW)], users_v)
    pltpu.sync_copy(movies_hbm.at[pl.ds(base, _BPW)], movies_v)
    cu = pltpu.async_copy(U_hbm.at[users_v], rows_u, sem)
    cm = pltpu.async_copy(M_hbm.at[movies_v], rows_m, sem)
    cu.wait()
    cm.wait()
    pltpu.sync_copy(rows_u, eu_hbm.at[pl.ds(base, _BPW)])
    pltpu.sync_copy(rows_m, em_hbm.at[pl.ds(base, _BPW)])


_sc_gather = functools.partial(
    pl.kernel,
    out_type=(
        jax.ShapeDtypeStruct((BATCH, NF), jnp.float32),
        jax.ShapeDtypeStruct((BATCH, NF), jnp.float32),
    ),
    mesh=plsc.VectorSubcoreMesh(core_axis_name="c", subcore_axis_name="s"),
    compiler_params=pltpu.CompilerParams(use_tc_tiling_on_sc=False),
    scratch_types=[
        pltpu.VMEM((_BPW,), jnp.int32),
        pltpu.VMEM((_BPW,), jnp.int32),
        pltpu.VMEM((_BPW, NF), jnp.float32),
        pltpu.VMEM((_BPW, NF), jnp.float32),
        pltpu.SemaphoreType.DMA,
    ],
)(_gather_body)


def _mlp_body(eu_ref, em_ref, w1u_ref, w1m_ref, b1_ref, w2_ref, b2_ref, out_ref):
    h = (jnp.dot(eu_ref[:], w1u_ref[:], preferred_element_type=jnp.float32)
         + jnp.dot(em_ref[:], w1m_ref[:], preferred_element_type=jnp.float32)
         + b1_ref[:])
    h = jnp.maximum(h, 0.0)
    o = jnp.dot(h, w2_ref[:], preferred_element_type=jnp.float32) + b2_ref[:]
    out_ref[:] = jax.nn.sigmoid(o) * 6.0 - 0.5


def kernel(users, movies, U, M, W1, b1, W2, b2):
    eu, em = _sc_gather(U, M, users.astype(jnp.int32), movies.astype(jnp.int32))
    w1u = W1[:, :NF].T
    w1m = W1[:, NF:].T
    out2d = pl.pallas_call(
        _mlp_body,
        out_shape=jax.ShapeDtypeStruct((BATCH, 1), jnp.float32),
    )(eu, em, w1u, w1m, b1[None, :], W2.T, b2[None, :])
    return out2d[:, 0]
